# Initial kernel scaffold; baseline (speedup 1.0000x reference)
import jax
import jax.numpy as jnp
from jax.experimental import pallas as pl


def kernel(D, X, A, W, b):
    raise NotImplementedError("write your pallas kernel here")


# speedup vs baseline: 1.7002x; 1.7002x over previous
import functools

import jax
import jax.numpy as jnp
from jax.experimental import pallas as pl

_N = 4096
_BM = 256


def _deg_xs_body(d_ref, x_ref, deg_ref, xs_ref):
    d = jnp.max(d_ref[...], axis=1, keepdims=True)
    deg_ref[...] = d
    xs_ref[...] = x_ref[...] * jax.lax.rsqrt(d)


def _conv_body(a_ref, xs_ref, deg_ref, w_ref, b_ref, o_ref):
    a = a_ref[...]
    ab = a > 0
    af = ab.astype(jnp.float32)
    agg = jnp.dot(af, xs_ref[...], preferred_element_type=jnp.float32)

    iota = jax.lax.broadcasted_iota(jnp.int32, a.shape, 1)
    masked = jnp.where(ab, iota, _N)
    first = jnp.min(masked, axis=1, keepdims=True)
    first = jnp.where(first >= _N, 0, first)
    onehot = (iota == first).astype(jnp.float32)
    dfirst = jnp.dot(onehot, deg_ref[...], preferred_element_type=jnp.float32)
    c = jax.lax.rsqrt(dfirst)

    z = jax.lax.dot_general(
        agg, w_ref[...], (((1,), (1,)), ((), ())),
        preferred_element_type=jnp.float32)
    z = z * c + b_ref[...]
    o_ref[...] = jnp.where(z >= 0.0, z, 0.01 * z)


@jax.jit
def kernel(D, X, A, W, b):
    n, in_ch = X.shape
    out_ch = W.shape[0]

    deg, xs = pl.pallas_call(
        _deg_xs_body,
        grid=(n // _BM,),
        in_specs=[
            pl.BlockSpec((_BM, n), lambda i: (i, 0)),
            pl.BlockSpec((_BM, in_ch), lambda i: (i, 0)),
        ],
        out_specs=[
            pl.BlockSpec((_BM, 1), lambda i: (i, 0)),
            pl.BlockSpec((_BM, in_ch), lambda i: (i, 0)),
        ],
        out_shape=[
            jax.ShapeDtypeStruct((n, 1), jnp.float32),
            jax.ShapeDtypeStruct((n, in_ch), jnp.float32),
        ],
    )(D, X)

    out = pl.pallas_call(
        _conv_body,
        grid=(n // _BM,),
        in_specs=[
            pl.BlockSpec((_BM, n), lambda i: (i, 0)),
            pl.BlockSpec((n, in_ch), lambda i: (0, 0)),
            pl.BlockSpec((n, 1), lambda i: (0, 0)),
            pl.BlockSpec((out_ch, in_ch), lambda i: (0, 0)),
            pl.BlockSpec((1, out_ch), lambda i: (0, 0)),
        ],
        out_specs=pl.BlockSpec((_BM, out_ch), lambda i: (i, 0)),
        out_shape=jax.ShapeDtypeStruct((n, out_ch), jnp.float32),
    )(A, xs, deg, W, b.reshape(1, out_ch))
    return out


# speedup vs baseline: 1.7440x; 1.0258x over previous
import functools

import jax
import jax.numpy as jnp
from jax.experimental import pallas as pl

_N = 4096
_BM = 256


def _deg_xs_body(d_ref, x_ref, deg_ref, xs_ref):
    d = jnp.max(d_ref[...], axis=1, keepdims=True)
    deg_ref[...] = d
    xs_ref[...] = (x_ref[...] * jax.lax.rsqrt(d)).astype(jnp.bfloat16)


def _conv_body(a_ref, xs_ref, deg_ref, w_ref, b_ref, o_ref):
    a = a_ref[...]
    ab = a > 0
    af = ab.astype(jnp.bfloat16)
    agg = jnp.dot(af, xs_ref[...], preferred_element_type=jnp.float32)

    iota = jax.lax.broadcasted_iota(jnp.int32, a.shape, 1)
    masked = jnp.where(ab, iota, _N)
    first = jnp.min(masked, axis=1, keepdims=True)
    first = jnp.where(first >= _N, 0, first)
    onehot = (iota == first).astype(jnp.float32)
    dfirst = jnp.dot(onehot, deg_ref[...], preferred_element_type=jnp.float32)
    c = jax.lax.rsqrt(dfirst)

    z = jax.lax.dot_general(
        agg, w_ref[...], (((1,), (1,)), ((), ())),
        preferred_element_type=jnp.float32)
    z = z * c + b_ref[...]
    o_ref[...] = jnp.where(z >= 0.0, z, 0.01 * z)


@jax.jit
def kernel(D, X, A, W, b):
    n, in_ch = X.shape
    out_ch = W.shape[0]

    deg, xs = pl.pallas_call(
        _deg_xs_body,
        grid=(n // _BM,),
        in_specs=[
            pl.BlockSpec((_BM, n), lambda i: (i, 0)),
            pl.BlockSpec((_BM, in_ch), lambda i: (i, 0)),
        ],
        out_specs=[
            pl.BlockSpec((_BM, 1), lambda i: (i, 0)),
            pl.BlockSpec((_BM, in_ch), lambda i: (i, 0)),
        ],
        out_shape=[
            jax.ShapeDtypeStruct((n, 1), jnp.float32),
            jax.ShapeDtypeStruct((n, in_ch), jnp.bfloat16),
        ],
    )(D, X)

    out = pl.pallas_call(
        _conv_body,
        grid=(n // _BM,),
        in_specs=[
            pl.BlockSpec((_BM, n), lambda i: (i, 0)),
            pl.BlockSpec((n, in_ch), lambda i: (0, 0)),
            pl.BlockSpec((n, 1), lambda i: (0, 0)),
            pl.BlockSpec((out_ch, in_ch), lambda i: (0, 0)),
            pl.BlockSpec((1, out_ch), lambda i: (0, 0)),
        ],
        out_specs=pl.BlockSpec((_BM, out_ch), lambda i: (i, 0)),
        out_shape=jax.ShapeDtypeStruct((n, out_ch), jnp.float32),
    )(A, xs, deg, W, b.reshape(1, out_ch))
    return out


# speedup vs baseline: 1.8754x; 1.0754x over previous
import functools

import jax
import jax.numpy as jnp
from jax.experimental import pallas as pl

_N = 4096
_BM = 512


def _deg_xs_body(d_ref, x_ref, deg_ref, xs_ref):
    d = jnp.max(d_ref[...], axis=1, keepdims=True)
    deg_ref[...] = d
    xs_ref[...] = (x_ref[...] * jax.lax.rsqrt(d)).astype(jnp.bfloat16)


def _conv_body(a_ref, xs_ref, deg_ref, w_ref, b_ref, o_ref):
    a = a_ref[...]
    ab = a > 0
    af = ab.astype(jnp.bfloat16)
    agg = jnp.dot(af, xs_ref[...], preferred_element_type=jnp.float32)

    iota = jax.lax.broadcasted_iota(jnp.int32, a.shape, 1)
    masked = jnp.where(ab, iota, _N)
    first = jnp.min(masked, axis=1, keepdims=True)
    first = jnp.where(first >= _N, 0, first)
    onehot = (iota == first).astype(jnp.float32)
    dfirst = jnp.dot(onehot, deg_ref[...], preferred_element_type=jnp.float32)
    c = jax.lax.rsqrt(dfirst)

    z = jax.lax.dot_general(
        agg, w_ref[...], (((1,), (1,)), ((), ())),
        preferred_element_type=jnp.float32)
    z = z * c + b_ref[...]
    o_ref[...] = jnp.where(z >= 0.0, z, 0.01 * z)


@jax.jit
def kernel(D, X, A, W, b):
    n, in_ch = X.shape
    out_ch = W.shape[0]

    deg, xs = pl.pallas_call(
        _deg_xs_body,
        grid=(n // _BM,),
        in_specs=[
            pl.BlockSpec((_BM, n), lambda i: (i, 0)),
            pl.BlockSpec((_BM, in_ch), lambda i: (i, 0)),
        ],
        out_specs=[
            pl.BlockSpec((_BM, 1), lambda i: (i, 0)),
            pl.BlockSpec((_BM, in_ch), lambda i: (i, 0)),
        ],
        out_shape=[
            jax.ShapeDtypeStruct((n, 1), jnp.float32),
            jax.ShapeDtypeStruct((n, in_ch), jnp.bfloat16),
        ],
    )(D, X)

    out = pl.pallas_call(
        _conv_body,
        grid=(n // _BM,),
        in_specs=[
            pl.BlockSpec((_BM, n), lambda i: (i, 0)),
            pl.BlockSpec((n, in_ch), lambda i: (0, 0)),
            pl.BlockSpec((n, 1), lambda i: (0, 0)),
            pl.BlockSpec((out_ch, in_ch), lambda i: (0, 0)),
            pl.BlockSpec((1, out_ch), lambda i: (0, 0)),
        ],
        out_specs=pl.BlockSpec((_BM, out_ch), lambda i: (i, 0)),
        out_shape=jax.ShapeDtypeStruct((n, out_ch), jnp.float32),
    )(A, xs, deg, W, b.reshape(1, out_ch))
    return out
